# baseline (speedup 1.0000x reference)
import jax
import jax.numpy as jnp
from jax import lax
from jax.experimental import pallas as pl
from jax.experimental.pallas import tpu as pltpu
from jax.experimental.pallas import tpu_sc as plsc

BATCH = 4096
FIELDS = 26
DIM = 16
DP1 = DIM + 1
NW = 32
BPW = BATCH // NW
NG = BPW // 16
VROWS = (1000000 * DP1) // 16


def _iota16():
    return lax.iota(jnp.int32, 16)


def _fm_body(t16_hbm, idxt_hbm, out_hbm,
             idx_v, g0a, g1a, g0b, g1b, win_a, win_b,
             accs, acc2s, out_v, sem_a, sem_b):
    wid = lax.axis_index("s") * 2 + lax.axis_index("c")
    zeros16 = jnp.zeros((16,), jnp.float32)
    iota = _iota16()

    pltpu.sync_copy(idxt_hbm.at[wid], idx_v)

    def zinit(bg, c):
        for d in range(DP1):
            accs[bg, d, :] = zeros16
            acc2s[bg, d, :] = zeros16
        return c
    lax.fori_loop(0, NG, zinit, 0)

    def stage_indices(j, g0s, g1s):
        jv = jnp.zeros((16,), jnp.int32) + j
        for g in range(NG):
            bvec = g * 16 + iota
            rv = plsc.load_gather(idx_v, [jv, bvec])
            g0 = (rv * DP1) >> 4
            g0s[pl.ds(g * 16, 16)] = g0
            g1s[pl.ds(g * 16, 16)] = g0 + 1

    def fire(g0s, g1s, win, sem):
        pltpu.async_copy(t16_hbm.at[g0s], win.at[0], sem)
        pltpu.async_copy(t16_hbm.at[g1s], win.at[1], sem)

    def wait(g0s, g1s, win, sem):
        pltpu.make_async_copy(t16_hbm.at[g0s], win.at[0], sem).wait()
        pltpu.make_async_copy(t16_hbm.at[g1s], win.at[1], sem).wait()

    def accumulate(j, win):
        jv = jnp.zeros((16,), jnp.int32) + j

        def bg_body(bg, c):
            bvec = bg * 16 + iota
            rv = plsc.load_gather(idx_v, [jv, bvec])
            ov = rv & 15
            for d in range(DP1):
                d0 = ov + d
                half = d0 >> 4
                pos = d0 & 15
                v = plsc.load_gather(win, [half, bvec, pos])
                accs[bg, d, :] = accs[bg, d, :] + v
                if d < DIM:
                    acc2s[bg, d, :] = acc2s[bg, d, :] + v * v
            return c

        lax.fori_loop(0, NG, bg_body, 0)

    stage_indices(0, g0a, g1a)
    fire(g0a, g1a, win_a, sem_a)

    def step(t, c):
        ja = 2 * t
        jb = 2 * t + 1
        stage_indices(jb, g0b, g1b)
        fire(g0b, g1b, win_b, sem_b)
        wait(g0a, g1a, win_a, sem_a)
        accumulate(ja, win_a)

        @pl.when(t < FIELDS // 2 - 1)
        def _():
            stage_indices(ja + 2, g0a, g1a)
            fire(g0a, g1a, win_a, sem_a)

        wait(g0b, g1b, win_b, sem_b)
        accumulate(jb, win_b)
        return c

    lax.fori_loop(0, FIELDS // 2, step, 0)

    def combine(bg, c):
        bvec = bg * 16 + iota
        plsc.store_scatter(out_v, [bvec, jnp.zeros((16,), jnp.int32)],
                           accs[bg, DIM, :])
        for d in range(DIM):
            s = accs[bg, d, :]
            val = 0.5 * (s * s - acc2s[bg, d, :])
            plsc.store_scatter(out_v, [bvec, jnp.zeros((16,), jnp.int32) + (d + 1)],
                               val)
        return c

    lax.fori_loop(0, NG, combine, 0)

    pltpu.sync_copy(out_v, out_hbm.at[pl.ds(wid * BPW, BPW)])


def kernel(inputs, kernel):
    t16 = kernel.reshape(VROWS, 16)
    idxt = inputs.reshape(NW, BPW, FIELDS).transpose(0, 2, 1)
    mesh = plsc.VectorSubcoreMesh(core_axis_name="c", subcore_axis_name="s")
    return pl.kernel(
        _fm_body,
        mesh=mesh,
        compiler_params=pltpu.CompilerParams(
            needs_layout_passes=False, use_tc_tiling_on_sc=False
        ),
        out_type=jax.ShapeDtypeStruct((BATCH, DP1), jnp.float32),
        scratch_types=[
            pltpu.VMEM((FIELDS, BPW), jnp.int32),
            pltpu.VMEM((BPW,), jnp.int32),
            pltpu.VMEM((BPW,), jnp.int32),
            pltpu.VMEM((BPW,), jnp.int32),
            pltpu.VMEM((BPW,), jnp.int32),
            pltpu.VMEM((2, BPW, 16), jnp.float32),
            pltpu.VMEM((2, BPW, 16), jnp.float32),
            pltpu.VMEM((NG, DP1, 16), jnp.float32),
            pltpu.VMEM((NG, DP1, 16), jnp.float32),
            pltpu.VMEM((BPW, DP1), jnp.float32),
            pltpu.SemaphoreType.DMA,
            pltpu.SemaphoreType.DMA,
        ],
    )(t16, idxt)


# speedup vs baseline: 1.6427x; 1.6427x over previous
import jax
import jax.numpy as jnp
from jax import lax
from jax.experimental import pallas as pl
from jax.experimental.pallas import tpu as pltpu
from jax.experimental.pallas import tpu_sc as plsc

BATCH = 4096
FIELDS = 26
DIM = 16
DP1 = DIM + 1
NW = 32
BPW = BATCH // NW
NG = BPW // 16
VOCAB = 1000000
RPB = 8192
NBLK = -(-VOCAB // RPB)
PACKED_ROWS = NBLK * (RPB // 4)


def _relayout_body(t_ref, out_ref):
    x = t_ref[...]
    xt = jnp.transpose(x, (1, 0))
    padded = jnp.concatenate(
        [xt, jnp.zeros((RPB, 32 - DP1), jnp.float32)], axis=1)
    p3 = padded.reshape(RPB // 4, 4, 32)
    out_ref[...] = jnp.concatenate([p3[:, q, :] for q in range(4)], axis=1)


def _relayout(table_t):
    return pl.pallas_call(
        _relayout_body,
        grid=(NBLK,),
        in_specs=[pl.BlockSpec((DP1, RPB), lambda i: (0, i))],
        out_specs=pl.BlockSpec((RPB // 4, 128), lambda i: (i, 0)),
        out_shape=jax.ShapeDtypeStruct((PACKED_ROWS, 128), jnp.float32),
    )(table_t)


def _fm_body(packed_hbm, idxt_hbm, out_hbm,
             idx_v, g_a, g_b, win, accs, acc2s, out_v, sem_a, sem_b):
    wid = lax.axis_index("s") * 2 + lax.axis_index("c")
    zeros16 = jnp.zeros((16,), jnp.float32)
    iota = lax.iota(jnp.int32, 16)

    pltpu.sync_copy(idxt_hbm.at[wid], idx_v)

    def zinit(bg, c):
        for d in range(DP1):
            accs[bg, d, pl.ds(0, 16)] = zeros16
            acc2s[bg, d, pl.ds(0, 16)] = zeros16
        return c
    lax.fori_loop(0, NG, zinit, 0)

    def stage_indices(j, gs):
        jv = jnp.zeros((16,), jnp.int32) + j
        for g in range(NG):
            rv = plsc.load_gather(idx_v, [jv, g * 16 + iota])
            gs[pl.ds(g * 16, 16)] = rv >> 2

    def fire(gs, buf, sem):
        pltpu.async_copy(packed_hbm.at[gs], win.at[buf], sem)

    def wait(gs, buf, sem):
        pltpu.make_async_copy(packed_hbm.at[gs], win.at[buf], sem).wait()

    def accumulate(j, buf):
        jv = jnp.zeros((16,), jnp.int32) + j
        bufv = jnp.zeros((16,), jnp.int32) + buf

        def bg_body(bg, c):
            bvec = bg * 16 + iota
            rv = plsc.load_gather(idx_v, [jv, bvec])
            base = (rv & 3) << 5
            for d in range(DP1):
                v = plsc.load_gather(win, [bufv, bvec, base + d])
                accs[bg, d, pl.ds(0, 16)] = accs[bg, d, pl.ds(0, 16)] + v
                if d < DIM:
                    acc2s[bg, d, pl.ds(0, 16)] = acc2s[bg, d, pl.ds(0, 16)] + v * v
            return c

        lax.fori_loop(0, NG, bg_body, 0)

    stage_indices(0, g_a)
    fire(g_a, 0, sem_a)

    def step(t, c):
        ja = 2 * t
        jb = 2 * t + 1
        stage_indices(jb, g_b)
        fire(g_b, 1, sem_b)
        wait(g_a, 0, sem_a)
        accumulate(ja, 0)

        @pl.when(t < FIELDS // 2 - 1)
        def _():
            stage_indices(ja + 2, g_a)
            fire(g_a, 0, sem_a)

        wait(g_b, 1, sem_b)
        accumulate(jb, 1)
        return c

    lax.fori_loop(0, FIELDS // 2, step, 0)

    def combine_rows(bg, c):
        bvec = bg * 16 + iota
        lin = accs[bg, DIM, pl.ds(0, 16)]
        plsc.store_scatter(out_v, [bvec, jnp.zeros((16,), jnp.int32)], lin)
        for d in range(DIM):
            s = accs[bg, d, pl.ds(0, 16)]
            val = 0.5 * (s * s - acc2s[bg, d, pl.ds(0, 16)])
            plsc.store_scatter(
                out_v, [bvec, jnp.zeros((16,), jnp.int32) + (d + 1)], val)
        return c

    lax.fori_loop(0, NG, combine_rows, 0)

    pltpu.sync_copy(out_v, out_hbm.at[pl.ds(wid * BPW, BPW)])


def kernel(inputs, kernel):
    packed = _relayout(kernel.T)
    idxt = inputs.reshape(NW, BPW, FIELDS).transpose(0, 2, 1)
    mesh = plsc.VectorSubcoreMesh(core_axis_name="c", subcore_axis_name="s")
    wide = pl.kernel(
        _fm_body,
        mesh=mesh,
        compiler_params=pltpu.CompilerParams(needs_layout_passes=False),
        out_type=jax.ShapeDtypeStruct((BATCH, 128), jnp.float32),
        scratch_types=[
            pltpu.VMEM((FIELDS, BPW), jnp.int32),
            pltpu.VMEM((BPW,), jnp.int32),
            pltpu.VMEM((BPW,), jnp.int32),
            pltpu.VMEM((2, BPW, 128), jnp.float32),
            pltpu.VMEM((NG, 24, 128), jnp.float32),
            pltpu.VMEM((NG, 24, 128), jnp.float32),
            pltpu.VMEM((BPW, 128), jnp.float32),
            pltpu.SemaphoreType.DMA,
            pltpu.SemaphoreType.DMA,
        ],
    )(packed, idxt)
    return wide[:, :DP1]


# speedup vs baseline: 2.5506x; 1.5527x over previous
import jax
import jax.numpy as jnp
from jax import lax
from jax.experimental import pallas as pl
from jax.experimental.pallas import tpu as pltpu
from jax.experimental.pallas import tpu_sc as plsc

BATCH = 4096
FIELDS = 26
DIM = 16
DP1 = DIM + 1
NW = 32
BPW = BATCH // NW
NG = BPW // 16
VOCAB = 1000000
RPB = 16384
NBLK = -(-VOCAB // RPB)
PACKED_ROWS = NBLK * (RPB // 4)
GROWS = PACKED_ROWS * 8


def _relayout_body(t_ref, out_ref):
    x = t_ref[...]
    xt = jnp.transpose(x, (1, 0))
    qq = RPB // 4
    for q in range(4):
        out_ref[:, pl.ds(32 * q, DP1)] = lax.slice(
            xt, (qq * q, 0), (qq * (q + 1), DP1))


def _relayout(table_t):
    return pl.pallas_call(
        _relayout_body,
        grid=(NBLK,),
        in_specs=[pl.BlockSpec((DP1, RPB), lambda i: (0, i))],
        out_specs=pl.BlockSpec((RPB // 4, 128), lambda i: (i, 0)),
        out_shape=jax.ShapeDtypeStruct((PACKED_ROWS, 128), jnp.float32),
    )(table_t)


def _fm_body(g16_hbm, idx_hbm, out_hbm,
             idx_v, g0a, g1a, g0b, g1b, win_a, win_b,
             accs, acc2s, out_v, sem_a, sem_b):
    wid = lax.axis_index("s") * 2 + lax.axis_index("c")
    zeros16 = jnp.zeros((16,), jnp.float32)
    iota = lax.iota(jnp.int32, 16)
    zerov = jnp.zeros((16,), jnp.int32)
    onev = zerov + 1

    pltpu.sync_copy(idx_hbm.at[pl.ds(wid * BPW, BPW)], idx_v)

    def zinit(bg, c):
        for d in range(DP1):
            accs[bg, d, :] = zeros16
            acc2s[bg, d, :] = zeros16
        return c
    lax.fori_loop(0, NG, zinit, 0)

    def stage_indices(j, g0s, g1s):
        jv = zerov + j
        for g in range(NG):
            rv = plsc.load_gather(idx_v, [g * 16 + iota, jv])
            i_b = rv >> 14
            rem = rv & (RPB - 1)
            q_b = rem >> 12
            p_b = rem & (RPB // 4 - 1)
            g0 = (i_b << 15) + (p_b << 3) + (q_b << 1)
            g0s[pl.ds(g * 16, 16)] = g0
            g1s[pl.ds(g * 16, 16)] = g0 + 1

    def fire(g0s, g1s, win, sem):
        pltpu.async_copy(g16_hbm.at[g0s], win.at[0], sem)
        pltpu.async_copy(g16_hbm.at[g1s], win.at[1], sem)

    def wait(g0s, g1s, win, sem):
        pltpu.make_async_copy(g16_hbm.at[g0s], win.at[0], sem).wait()
        pltpu.make_async_copy(g16_hbm.at[g1s], win.at[1], sem).wait()

    def accumulate(j, win):
        def bg_body(bg, c):
            bvec = bg * 16 + iota
            lin = plsc.load_gather(win, [onev, bvec, zerov])
            accs[bg, DIM, :] = accs[bg, DIM, :] + lin
            for d in range(DIM):
                v = plsc.load_gather(win, [zerov, bvec, zerov + d])
                accs[bg, d, :] = accs[bg, d, :] + v
                acc2s[bg, d, :] = acc2s[bg, d, :] + v * v
            return c

        lax.fori_loop(0, NG, bg_body, 0)

    stage_indices(0, g0a, g1a)
    fire(g0a, g1a, win_a, sem_a)

    def step(t, c):
        ja = 2 * t
        jb = 2 * t + 1
        stage_indices(jb, g0b, g1b)
        fire(g0b, g1b, win_b, sem_b)
        wait(g0a, g1a, win_a, sem_a)
        accumulate(ja, win_a)

        @pl.when(t < FIELDS // 2 - 1)
        def _():
            stage_indices(ja + 2, g0a, g1a)
            fire(g0a, g1a, win_a, sem_a)

        wait(g0b, g1b, win_b, sem_b)
        accumulate(jb, win_b)
        return c

    lax.fori_loop(0, FIELDS // 2, step, 0)

    def combine_rows(bg, c):
        bvec = bg * 16 + iota
        plsc.store_scatter(out_v, [bvec, zerov], accs[bg, DIM, :])
        for d in range(DIM):
            s = accs[bg, d, :]
            val = 0.5 * (s * s - acc2s[bg, d, :])
            plsc.store_scatter(out_v, [bvec, zerov + (d + 1)], val)
        return c

    lax.fori_loop(0, NG, combine_rows, 0)

    pltpu.sync_copy(out_v, out_hbm.at[pl.ds(wid * BPW, BPW)])


def kernel(inputs, kernel):
    packed = _relayout(kernel.T)
    g16 = packed.reshape(GROWS, 16)
    mesh = plsc.VectorSubcoreMesh(core_axis_name="c", subcore_axis_name="s")
    return pl.kernel(
        _fm_body,
        mesh=mesh,
        compiler_params=pltpu.CompilerParams(
            needs_layout_passes=False, use_tc_tiling_on_sc=False
        ),
        out_type=jax.ShapeDtypeStruct((BATCH, DP1), jnp.float32),
        scratch_types=[
            pltpu.VMEM((BPW, FIELDS), jnp.int32),
            pltpu.VMEM((BPW,), jnp.int32),
            pltpu.VMEM((BPW,), jnp.int32),
            pltpu.VMEM((BPW,), jnp.int32),
            pltpu.VMEM((BPW,), jnp.int32),
            pltpu.VMEM((2, BPW, 16), jnp.float32),
            pltpu.VMEM((2, BPW, 16), jnp.float32),
            pltpu.VMEM((NG, DP1, 16), jnp.float32),
            pltpu.VMEM((NG, DP1, 16), jnp.float32),
            pltpu.VMEM((BPW, DP1), jnp.float32),
            pltpu.SemaphoreType.DMA,
            pltpu.SemaphoreType.DMA,
        ],
    )(g16, inputs)


# speedup vs baseline: 5.2476x; 2.0574x over previous
import jax
import jax.numpy as jnp
from jax import lax
from jax.experimental import pallas as pl
from jax.experimental.pallas import tpu as pltpu
from jax.experimental.pallas import tpu_sc as plsc

BATCH = 4096
FIELDS = 26
DIM = 16
DP1 = DIM + 1
NW = 32
BPW = BATCH // NW
NG = BPW // 16
VOCAB = 1000000
LPB = 2048
RPB = 7 * LPB
NBLK = -(-VOCAB // RPB)
PACKED_LINES = NBLK * LPB
GROWS = PACKED_LINES * 8


def _relayout_body(t_ref, out_ref):
    x = t_ref[...]
    y = jnp.concatenate(
        [lax.slice(x, (0, LPB * k), (DP1, LPB * (k + 1))) for k in range(7)],
        axis=0)
    yt = jnp.transpose(y, (1, 0))
    out_ref[:, pl.ds(0, 7 * DP1)] = yt


def _relayout(table_t):
    return pl.pallas_call(
        _relayout_body,
        grid=(NBLK,),
        in_specs=[pl.BlockSpec((DP1, RPB), lambda i: (0, i))],
        out_specs=pl.BlockSpec((LPB, 128), lambda i: (i, 0)),
        out_shape=jax.ShapeDtypeStruct((PACKED_LINES, 128), jnp.float32),
    )(table_t)


def _split_rpk(rv):
    q = rv >> 11
    i_b = (q * 9363) >> 16
    k_b = q - i_b * 7
    p_b = rv & 2047
    return i_b, k_b, p_b


def _fm_body(g16_hbm, idx_hbm, out_hbm,
             idx_v, g0a, g1a, g0b, g1b, win_a, win_b,
             accs, acc2s, out_v, sem_a, sem_b):
    wid = lax.axis_index("s") * 2 + lax.axis_index("c")
    zeros16 = jnp.zeros((16,), jnp.float32)
    iota = lax.iota(jnp.int32, 16)
    zerov = jnp.zeros((16,), jnp.int32)

    pltpu.sync_copy(idx_hbm.at[pl.ds(wid * BPW, BPW)], idx_v)

    def zinit(bg, c):
        for d in range(DP1):
            accs[bg, d, :] = zeros16
            acc2s[bg, d, :] = zeros16
        return c
    lax.fori_loop(0, NG, zinit, 0)

    def stage_indices(j, g0s, g1s):
        jv = zerov + j
        for g in range(NG):
            rv = plsc.load_gather(idx_v, [g * 16 + iota, jv])
            i_b, k_b, p_b = _split_rpk(rv)
            g0 = (i_b << 14) + (p_b << 3) + k_b
            g0s[pl.ds(g * 16, 16)] = g0
            g1s[pl.ds(g * 16, 16)] = g0 + 1

    def fire(g0s, g1s, win, sem):
        pltpu.async_copy(g16_hbm.at[g0s], win.at[0], sem)
        pltpu.async_copy(g16_hbm.at[g1s], win.at[1], sem)

    def wait(g0s, g1s, win, sem):
        pltpu.make_async_copy(g16_hbm.at[g0s], win.at[0], sem).wait()
        pltpu.make_async_copy(g16_hbm.at[g1s], win.at[1], sem).wait()

    def accumulate(j, win):
        jv = zerov + j

        def bg_body(bg, c):
            bvec = bg * 16 + iota
            rv = plsc.load_gather(idx_v, [bvec, jv])
            _, k_b, _ = _split_rpk(rv)
            for d in range(DP1):
                d0 = k_b + d
                half = d0 >> 4
                pos = d0 & 15
                v = plsc.load_gather(win, [half, bvec, pos])
                accs[bg, d, :] = accs[bg, d, :] + v
                if d < DIM:
                    acc2s[bg, d, :] = acc2s[bg, d, :] + v * v
            return c

        lax.fori_loop(0, NG, bg_body, 0)

    stage_indices(0, g0a, g1a)
    fire(g0a, g1a, win_a, sem_a)

    def step(t, c):
        ja = 2 * t
        jb = 2 * t + 1
        stage_indices(jb, g0b, g1b)
        fire(g0b, g1b, win_b, sem_b)
        wait(g0a, g1a, win_a, sem_a)
        accumulate(ja, win_a)

        @pl.when(t < FIELDS // 2 - 1)
        def _():
            stage_indices(ja + 2, g0a, g1a)
            fire(g0a, g1a, win_a, sem_a)

        wait(g0b, g1b, win_b, sem_b)
        accumulate(jb, win_b)
        return c

    lax.fori_loop(0, FIELDS // 2, step, 0)

    def combine_rows(bg, c):
        bvec = bg * 16 + iota
        plsc.store_scatter(out_v, [bvec, zerov], accs[bg, DIM, :])
        for d in range(DIM):
            s = accs[bg, d, :]
            val = 0.5 * (s * s - acc2s[bg, d, :])
            plsc.store_scatter(out_v, [bvec, zerov + (d + 1)], val)
        return c

    lax.fori_loop(0, NG, combine_rows, 0)

    pltpu.sync_copy(out_v, out_hbm.at[pl.ds(wid * BPW, BPW)])


def kernel(inputs, kernel):
    packed = _relayout(kernel.T)
    g16 = packed.reshape(GROWS, 16)
    mesh = plsc.VectorSubcoreMesh(core_axis_name="c", subcore_axis_name="s")
    return pl.kernel(
        _fm_body,
        mesh=mesh,
        compiler_params=pltpu.CompilerParams(
            needs_layout_passes=False, use_tc_tiling_on_sc=False
        ),
        out_type=jax.ShapeDtypeStruct((BATCH, DP1), jnp.float32),
        scratch_types=[
            pltpu.VMEM((BPW, FIELDS), jnp.int32),
            pltpu.VMEM((BPW,), jnp.int32),
            pltpu.VMEM((BPW,), jnp.int32),
            pltpu.VMEM((BPW,), jnp.int32),
            pltpu.VMEM((BPW,), jnp.int32),
            pltpu.VMEM((2, BPW, 16), jnp.float32),
            pltpu.VMEM((2, BPW, 16), jnp.float32),
            pltpu.VMEM((NG, DP1, 16), jnp.float32),
            pltpu.VMEM((NG, DP1, 16), jnp.float32),
            pltpu.VMEM((BPW, DP1), jnp.float32),
            pltpu.SemaphoreType.DMA,
            pltpu.SemaphoreType.DMA,
        ],
    )(g16, inputs)


# speedup vs baseline: 6.1261x; 1.1674x over previous
import jax
import jax.numpy as jnp
from jax import lax
from jax.experimental import pallas as pl
from jax.experimental.pallas import tpu as pltpu
from jax.experimental.pallas import tpu_sc as plsc

BATCH = 4096
FIELDS = 26
DIM = 16
DP1 = DIM + 1
NW = 32
BPW = BATCH // NW
NG = BPW // 16
VOCAB = 1000000
LPB = 4096
RPB = 7 * LPB
NBLK = -(-VOCAB // RPB)
PACKED_LINES = NBLK * LPB
GROWS = PACKED_LINES * 8


def _relayout_body(t_ref, out_ref):
    x = t_ref[...]
    y = jnp.concatenate(
        [lax.slice(x, (0, LPB * k), (DP1, LPB * (k + 1))) for k in range(7)],
        axis=0)
    yt = jnp.transpose(y, (1, 0))
    out_ref[:, pl.ds(0, 7 * DP1)] = yt


def _relayout(table_t):
    return pl.pallas_call(
        _relayout_body,
        grid=(NBLK,),
        in_specs=[pl.BlockSpec((DP1, RPB), lambda i: (0, i))],
        out_specs=pl.BlockSpec((LPB, 128), lambda i: (i, 0)),
        out_shape=jax.ShapeDtypeStruct((PACKED_LINES, 128), jnp.float32),
    )(table_t)


def _split_rpk(rv):
    q = rv >> 12
    i_b = (q * 9363) >> 16
    k_b = q - i_b * 7
    p_b = rv & 4095
    return i_b, k_b, p_b


def _fm_body(g16_hbm, idx_hbm, out_hbm,
             idx_v, g0f, g1f, win_all, accs, acc2s, out_v, sem):
    wid = lax.axis_index("s") * 2 + lax.axis_index("c")
    zeros16 = jnp.zeros((16,), jnp.float32)
    iota = lax.iota(jnp.int32, 16)
    zerov = jnp.zeros((16,), jnp.int32)

    pltpu.sync_copy(idx_hbm.at[pl.ds(wid * BPW, BPW)], idx_v)

    def zinit(bg, c):
        for d in range(DP1):
            accs[bg, d, :] = zeros16
            acc2s[bg, d, :] = zeros16
        return c
    lax.fori_loop(0, NG, zinit, 0)

    def stage_fire(j, c):
        jv = zerov + j
        for g in range(NG):
            rv = plsc.load_gather(idx_v, [g * 16 + iota, jv])
            i_b, k_b, p_b = _split_rpk(rv)
            g0 = (i_b << 15) + (p_b << 3) + k_b
            g0f[pl.ds(j * BPW + g * 16, 16)] = g0
            g1f[pl.ds(j * BPW + g * 16, 16)] = g0 + 1
        pltpu.async_copy(g16_hbm.at[g0f.at[pl.ds(j * BPW, BPW)]],
                         win_all.at[j, 0], sem)
        pltpu.async_copy(g16_hbm.at[g1f.at[pl.ds(j * BPW, BPW)]],
                         win_all.at[j, 1], sem)
        return c

    lax.fori_loop(0, FIELDS, stage_fire, 0)

    def accumulate(j):
        jv = zerov + j

        def bg_body(bg, c):
            bvec = bg * 16 + iota
            rv = plsc.load_gather(idx_v, [bvec, jv])
            _, k_b, _ = _split_rpk(rv)
            for d in range(DP1):
                d0 = k_b + d
                half = d0 >> 4
                pos = d0 & 15
                v = plsc.load_gather(win_all, [jv, half, bvec, pos])
                accs[bg, d, :] = accs[bg, d, :] + v
                if d < DIM:
                    acc2s[bg, d, :] = acc2s[bg, d, :] + v * v
            return c

        lax.fori_loop(0, NG, bg_body, 0)

    def drain_acc(j, c):
        pltpu.make_async_copy(g16_hbm.at[g0f.at[pl.ds(j * BPW, BPW)]],
                              win_all.at[j, 0], sem).wait()
        pltpu.make_async_copy(g16_hbm.at[g1f.at[pl.ds(j * BPW, BPW)]],
                              win_all.at[j, 1], sem).wait()
        accumulate(j)
        return c

    lax.fori_loop(0, FIELDS, drain_acc, 0)

    def combine_rows(bg, c):
        bvec = bg * 16 + iota
        plsc.store_scatter(out_v, [bvec, zerov], accs[bg, DIM, :])
        for d in range(DIM):
            s = accs[bg, d, :]
            val = 0.5 * (s * s - acc2s[bg, d, :])
            plsc.store_scatter(out_v, [bvec, zerov + (d + 1)], val)
        return c

    lax.fori_loop(0, NG, combine_rows, 0)

    pltpu.sync_copy(out_v, out_hbm.at[pl.ds(wid * BPW, BPW)])


def kernel(inputs, kernel):
    packed = _relayout(kernel.T)
    g16 = packed.reshape(GROWS, 16)
    mesh = plsc.VectorSubcoreMesh(core_axis_name="c", subcore_axis_name="s")
    return pl.kernel(
        _fm_body,
        mesh=mesh,
        compiler_params=pltpu.CompilerParams(
            needs_layout_passes=False, use_tc_tiling_on_sc=False
        ),
        out_type=jax.ShapeDtypeStruct((BATCH, DP1), jnp.float32),
        scratch_types=[
            pltpu.VMEM((BPW, FIELDS), jnp.int32),
            pltpu.VMEM((FIELDS * BPW,), jnp.int32),
            pltpu.VMEM((FIELDS * BPW,), jnp.int32),
            pltpu.VMEM((FIELDS, 2, BPW, 16), jnp.float32),
            pltpu.VMEM((NG, DP1, 16), jnp.float32),
            pltpu.VMEM((NG, DP1, 16), jnp.float32),
            pltpu.VMEM((BPW, DP1), jnp.float32),
            pltpu.SemaphoreType.DMA,
        ],
    )(g16, inputs)


# speedup vs baseline: 7.7914x; 1.2718x over previous
import jax
import jax.numpy as jnp
from jax import lax
from jax.experimental import pallas as pl
from jax.experimental.pallas import tpu as pltpu
from jax.experimental.pallas import tpu_sc as plsc

BATCH = 4096
FIELDS = 26
DIM = 16
DP1 = DIM + 1
NW = 32
BPW = BATCH // NW
NG = BPW // 16
VOCAB = 1000000
LPB = 4096
RPB = 7 * LPB
NBLK = -(-VOCAB // RPB)
PACKED_LINES = NBLK * LPB
GROWS = PACKED_LINES * 8


def _relayout_body(t_ref, out_ref):
    x = t_ref[...]
    y = jnp.concatenate(
        [lax.slice(x, (0, LPB * k), (DP1, LPB * (k + 1))) for k in range(7)],
        axis=0)
    yt = jnp.transpose(y, (1, 0))
    out_ref[:, pl.ds(0, 7 * DP1)] = yt


def _relayout(table_t):
    return pl.pallas_call(
        _relayout_body,
        grid=(NBLK,),
        in_specs=[pl.BlockSpec((DP1, RPB), lambda i: (0, i))],
        out_specs=pl.BlockSpec((LPB, 128), lambda i: (i, 0)),
        out_shape=jax.ShapeDtypeStruct((PACKED_LINES, 128), jnp.float32),
    )(table_t)


def _split_rpk(rv):
    q = rv >> 12
    i_b = (q * 9363) >> 16
    k_b = q - i_b * 7
    p_b = rv & 4095
    return i_b, k_b, p_b


def _fm_body(g16_hbm, idx_hbm, out_hbm,
             idx_v, g0f, g1f, win_all, out_v, sem):
    wid = lax.axis_index("s") * 2 + lax.axis_index("c")
    zeros16 = jnp.zeros((16,), jnp.float32)
    iota = lax.iota(jnp.int32, 16)
    zerov = jnp.zeros((16,), jnp.int32)

    pltpu.sync_copy(idx_hbm.at[pl.ds(wid * BPW, BPW)], idx_v)

    def stage_fire(j, c):
        jv = zerov + j
        for g in range(NG):
            rv = plsc.load_gather(idx_v, [g * 16 + iota, jv])
            i_b, k_b, p_b = _split_rpk(rv)
            g0 = (i_b << 15) + (p_b << 3) + k_b
            g0f[pl.ds(j * BPW + g * 16, 16)] = g0
            g1f[pl.ds(j * BPW + g * 16, 16)] = g0 + 1
        pltpu.async_copy(g16_hbm.at[g0f.at[pl.ds(j * BPW, BPW)]],
                         win_all.at[j, 0], sem)
        pltpu.async_copy(g16_hbm.at[g1f.at[pl.ds(j * BPW, BPW)]],
                         win_all.at[j, 1], sem)
        return c

    lax.fori_loop(0, FIELDS, stage_fire, 0)

    def drain(j, c):
        pltpu.make_async_copy(g16_hbm.at[g0f.at[pl.ds(j * BPW, BPW)]],
                              win_all.at[j, 0], sem).wait()
        pltpu.make_async_copy(g16_hbm.at[g1f.at[pl.ds(j * BPW, BPW)]],
                              win_all.at[j, 1], sem).wait()
        return c

    lax.fori_loop(0, FIELDS, drain, 0)

    def bg_body(bg, c):
        bvec = bg * 16 + iota

        def jbody(j, carry):
            jv = zerov + j
            rv = plsc.load_gather(idx_v, [bvec, jv])
            _, k_b, _ = _split_rpk(rv)
            vs = []
            out = []
            for d in range(DP1):
                d0 = k_b + d
                half = d0 >> 4
                pos = d0 & 15
                v = plsc.load_gather(win_all, [jv, half, bvec, pos])
                vs.append(v)
                out.append(carry[d] + v)
            for d in range(DIM):
                out.append(carry[DP1 + d] + vs[d] * vs[d])
            return tuple(out)

        init = tuple(zeros16 for _ in range(DP1 + DIM))
        acc = lax.fori_loop(0, FIELDS, jbody, init)

        plsc.store_scatter(out_v, [bvec, zerov], acc[DIM])
        for d in range(DIM):
            s = acc[d]
            val = 0.5 * (s * s - acc[DP1 + d])
            plsc.store_scatter(out_v, [bvec, zerov + (d + 1)], val)
        return c

    lax.fori_loop(0, NG, bg_body, 0)

    pltpu.sync_copy(out_v, out_hbm.at[pl.ds(wid * BPW, BPW)])


def kernel(inputs, kernel):
    packed = _relayout(kernel.T)
    g16 = packed.reshape(GROWS, 16)
    mesh = plsc.VectorSubcoreMesh(core_axis_name="c", subcore_axis_name="s")
    return pl.kernel(
        _fm_body,
        mesh=mesh,
        compiler_params=pltpu.CompilerParams(
            needs_layout_passes=False, use_tc_tiling_on_sc=False
        ),
        out_type=jax.ShapeDtypeStruct((BATCH, DP1), jnp.float32),
        scratch_types=[
            pltpu.VMEM((BPW, FIELDS), jnp.int32),
            pltpu.VMEM((FIELDS * BPW,), jnp.int32),
            pltpu.VMEM((FIELDS * BPW,), jnp.int32),
            pltpu.VMEM((FIELDS, 2, BPW, 16), jnp.float32),
            pltpu.VMEM((BPW, DP1), jnp.float32),
            pltpu.SemaphoreType.DMA,
        ],
    )(g16, inputs)


# speedup vs baseline: 8.3291x; 1.0690x over previous
import jax
import jax.numpy as jnp
from jax import lax
from jax.experimental import pallas as pl
from jax.experimental.pallas import tpu as pltpu
from jax.experimental.pallas import tpu_sc as plsc

BATCH = 4096
FIELDS = 26
DIM = 16
DP1 = DIM + 1
NW = 32
BPW = BATCH // NW
NG = BPW // 16
VOCAB = 1000000
LPB = 8192
RPB = 7 * LPB
NBLK = -(-VOCAB // RPB)
PACKED_LINES = NBLK * LPB
GROWS = PACKED_LINES * 8


def _relayout_body(t_ref, out_ref):
    x = t_ref[...]
    y = jnp.concatenate(
        [lax.slice(x, (0, LPB * k), (DP1, LPB * (k + 1))) for k in range(7)],
        axis=0)
    yt = jnp.transpose(y, (1, 0))
    out_ref[:, pl.ds(0, 7 * DP1)] = yt


def _relayout(table_t):
    return pl.pallas_call(
        _relayout_body,
        grid=(NBLK,),
        in_specs=[pl.BlockSpec((DP1, RPB), lambda i: (0, i))],
        out_specs=pl.BlockSpec((LPB, 128), lambda i: (i, 0)),
        out_shape=jax.ShapeDtypeStruct((PACKED_LINES, 128), jnp.float32),
    )(table_t)


def _split_rpk(rv):
    q = rv >> 13
    i_b = (q * 9363) >> 16
    k_b = q - i_b * 7
    p_b = rv & 8191
    return i_b, k_b, p_b


def _fm_body(g16_hbm, idx_hbm, out_hbm,
             idx_v, g0f, g1f, win_all, out_v, sem):
    wid = lax.axis_index("s") * 2 + lax.axis_index("c")
    zeros16 = jnp.zeros((16,), jnp.float32)
    iota = lax.iota(jnp.int32, 16)
    zerov = jnp.zeros((16,), jnp.int32)

    pltpu.sync_copy(idx_hbm.at[pl.ds(wid * BPW, BPW)], idx_v)

    def stage_fire(j, c):
        jv = zerov + j
        for g in range(NG):
            rv = plsc.load_gather(idx_v, [g * 16 + iota, jv])
            i_b, k_b, p_b = _split_rpk(rv)
            g0 = (i_b << 16) + (p_b << 3) + k_b
            g0f[pl.ds(j * BPW + g * 16, 16)] = g0
            g1f[pl.ds(j * BPW + g * 16, 16)] = g0 + 1
        pltpu.async_copy(g16_hbm.at[g0f.at[pl.ds(j * BPW, BPW)]],
                         win_all.at[j, 0], sem)
        pltpu.async_copy(g16_hbm.at[g1f.at[pl.ds(j * BPW, BPW)]],
                         win_all.at[j, 1], sem)
        return c

    lax.fori_loop(0, FIELDS, stage_fire, 0)

    def drain(j, c):
        pltpu.make_async_copy(g16_hbm.at[g0f.at[pl.ds(j * BPW, BPW)]],
                              win_all.at[j, 0], sem).wait()
        pltpu.make_async_copy(g16_hbm.at[g1f.at[pl.ds(j * BPW, BPW)]],
                              win_all.at[j, 1], sem).wait()
        return c

    lax.fori_loop(0, FIELDS, drain, 0)

    def bg_body(bg, c):
        bvec = bg * 16 + iota

        def jbody(j, carry):
            jv = zerov + j
            rv = plsc.load_gather(idx_v, [bvec, jv])
            _, k_b, _ = _split_rpk(rv)
            vs = []
            out = []
            for d in range(DP1):
                d0 = k_b + d
                half = d0 >> 4
                pos = d0 & 15
                v = plsc.load_gather(win_all, [jv, half, bvec, pos])
                vs.append(v)
                out.append(carry[d] + v)
            for d in range(DIM):
                out.append(carry[DP1 + d] + vs[d] * vs[d])
            return tuple(out)

        init = tuple(zeros16 for _ in range(DP1 + DIM))
        acc = lax.fori_loop(0, FIELDS, jbody, init)

        plsc.store_scatter(out_v, [bvec, zerov], acc[DIM])
        for d in range(DIM):
            s = acc[d]
            val = 0.5 * (s * s - acc[DP1 + d])
            plsc.store_scatter(out_v, [bvec, zerov + (d + 1)], val)
        return c

    lax.fori_loop(0, NG, bg_body, 0)

    pltpu.sync_copy(out_v, out_hbm.at[pl.ds(wid * BPW, BPW)])


def kernel(inputs, kernel):
    packed = _relayout(kernel.T)
    g16 = packed.reshape(GROWS, 16)
    mesh = plsc.VectorSubcoreMesh(core_axis_name="c", subcore_axis_name="s")
    return pl.kernel(
        _fm_body,
        mesh=mesh,
        compiler_params=pltpu.CompilerParams(
            needs_layout_passes=False, use_tc_tiling_on_sc=False
        ),
        out_type=jax.ShapeDtypeStruct((BATCH, DP1), jnp.float32),
        scratch_types=[
            pltpu.VMEM((BPW, FIELDS), jnp.int32),
            pltpu.VMEM((FIELDS * BPW,), jnp.int32),
            pltpu.VMEM((FIELDS * BPW,), jnp.int32),
            pltpu.VMEM((FIELDS, 2, BPW, 16), jnp.float32),
            pltpu.VMEM((BPW, DP1), jnp.float32),
            pltpu.SemaphoreType.DMA,
        ],
    )(g16, inputs)


# speedup vs baseline: 8.4020x; 1.0088x over previous
import jax
import jax.numpy as jnp
from jax import lax
from jax.experimental import pallas as pl
from jax.experimental.pallas import tpu as pltpu
from jax.experimental.pallas import tpu_sc as plsc

BATCH = 4096
FIELDS = 26
DIM = 16
DP1 = DIM + 1
NW = 32
BPW = BATCH // NW
NG = BPW // 16
VOCAB = 1000000
LPB = 16384
RPB = 7 * LPB
NBLK = -(-VOCAB // RPB)
PACKED_LINES = NBLK * LPB
GROWS = PACKED_LINES * 8


def _relayout_body(t_ref, out_ref):
    x = t_ref[...]
    y = jnp.concatenate(
        [lax.slice(x, (0, LPB * k), (DP1, LPB * (k + 1))) for k in range(7)],
        axis=0)
    yt = jnp.transpose(y, (1, 0))
    out_ref[:, pl.ds(0, 7 * DP1)] = yt


def _relayout(table_t):
    return pl.pallas_call(
        _relayout_body,
        grid=(NBLK,),
        in_specs=[pl.BlockSpec((DP1, RPB), lambda i: (0, i))],
        out_specs=pl.BlockSpec((LPB, 128), lambda i: (i, 0)),
        out_shape=jax.ShapeDtypeStruct((PACKED_LINES, 128), jnp.float32),
    )(table_t)


def _split_rpk(rv):
    q = rv >> 14
    i_b = (q * 9363) >> 16
    k_b = q - i_b * 7
    p_b = rv & 16383
    return i_b, k_b, p_b


def _fm_body(g16_hbm, idx_hbm, out_hbm,
             idx_v, g0f, g1f, win_all, out_v, sem):
    wid = lax.axis_index("s") * 2 + lax.axis_index("c")
    zeros16 = jnp.zeros((16,), jnp.float32)
    iota = lax.iota(jnp.int32, 16)
    zerov = jnp.zeros((16,), jnp.int32)

    pltpu.sync_copy(idx_hbm.at[pl.ds(wid * BPW, BPW)], idx_v)

    def stage_fire(j, c):
        jv = zerov + j
        for g in range(NG):
            rv = plsc.load_gather(idx_v, [g * 16 + iota, jv])
            i_b, k_b, p_b = _split_rpk(rv)
            g0 = (i_b << 17) + (p_b << 3) + k_b
            g0f[pl.ds(j * BPW + g * 16, 16)] = g0
            g1f[pl.ds(j * BPW + g * 16, 16)] = g0 + 1
        pltpu.async_copy(g16_hbm.at[g0f.at[pl.ds(j * BPW, BPW)]],
                         win_all.at[j, 0], sem)
        pltpu.async_copy(g16_hbm.at[g1f.at[pl.ds(j * BPW, BPW)]],
                         win_all.at[j, 1], sem)
        return c

    lax.fori_loop(0, FIELDS, stage_fire, 0)

    def drain(j, c):
        pltpu.make_async_copy(g16_hbm.at[g0f.at[pl.ds(j * BPW, BPW)]],
                              win_all.at[j, 0], sem).wait()
        pltpu.make_async_copy(g16_hbm.at[g1f.at[pl.ds(j * BPW, BPW)]],
                              win_all.at[j, 1], sem).wait()
        return c

    lax.fori_loop(0, FIELDS, drain, 0)

    def bg_body(bg, c):
        bvec = bg * 16 + iota

        def jbody(j, carry):
            jv = zerov + j
            rv = plsc.load_gather(idx_v, [bvec, jv])
            _, k_b, _ = _split_rpk(rv)
            vs = []
            out = []
            for d in range(DP1):
                d0 = k_b + d
                half = d0 >> 4
                pos = d0 & 15
                v = plsc.load_gather(win_all, [jv, half, bvec, pos])
                vs.append(v)
                out.append(carry[d] + v)
            for d in range(DIM):
                out.append(carry[DP1 + d] + vs[d] * vs[d])
            return tuple(out)

        init = tuple(zeros16 for _ in range(DP1 + DIM))
        acc = lax.fori_loop(0, FIELDS, jbody, init)

        plsc.store_scatter(out_v, [bvec, zerov], acc[DIM])
        for d in range(DIM):
            s = acc[d]
            val = 0.5 * (s * s - acc[DP1 + d])
            plsc.store_scatter(out_v, [bvec, zerov + (d + 1)], val)
        return c

    lax.fori_loop(0, NG, bg_body, 0)

    pltpu.sync_copy(out_v, out_hbm.at[pl.ds(wid * BPW, BPW)])


def kernel(inputs, kernel):
    packed = _relayout(kernel.T)
    g16 = packed.reshape(GROWS, 16)
    mesh = plsc.VectorSubcoreMesh(core_axis_name="c", subcore_axis_name="s")
    return pl.kernel(
        _fm_body,
        mesh=mesh,
        compiler_params=pltpu.CompilerParams(
            needs_layout_passes=False, use_tc_tiling_on_sc=False
        ),
        out_type=jax.ShapeDtypeStruct((BATCH, DP1), jnp.float32),
        scratch_types=[
            pltpu.VMEM((BPW, FIELDS), jnp.int32),
            pltpu.VMEM((FIELDS * BPW,), jnp.int32),
            pltpu.VMEM((FIELDS * BPW,), jnp.int32),
            pltpu.VMEM((FIELDS, 2, BPW, 16), jnp.float32),
            pltpu.VMEM((BPW, DP1), jnp.float32),
            pltpu.SemaphoreType.DMA,
        ],
    )(g16, inputs)
